# baseline (speedup 1.0000x reference)
import functools

import jax
import jax.numpy as jnp
from jax import lax
from jax.experimental import pallas as pl
from jax.experimental.pallas import tpu as pltpu
from jax.experimental.pallas import tpu_sc as plsc

NUM_EXPERTS = 8
MOE_DIM = 768
FF_DIM = 1024
BEH_DIM = 64
NUM_BEH = 16
BEH_ROWS = 32
N_TOK = 2048
TILE = 128
NTILES = 24
PADDED = NTILES * TILE

_NC, _NS = 2, 16
_NW = _NC * _NS
_MESH = plsc.VectorSubcoreMesh(core_axis_name="c", subcore_axis_name="s",
                               num_cores=_NC, num_subcores=_NS)


_TOK_W = N_TOK // _NW


@functools.partial(
    pl.kernel, mesh=_MESH,
    out_type=jax.ShapeDtypeStruct((PADDED, MOE_DIM + 128), jnp.float32),
    scratch_types=[pltpu.VMEM((_TOK_W,), jnp.int32),
                   pltpu.VMEM((_TOK_W, MOE_DIM + 128), jnp.float32),
                   pltpu.SemaphoreType.DMA],
)
def _sc_scatter_pad(dst_hbm, hid_hbm, xh_hbm, idx_v, rows_v, sem):
    wid = lax.axis_index("s") * _NC + lax.axis_index("c")
    base = wid * _TOK_W
    pltpu.sync_copy(dst_hbm.at[pl.ds(base, _TOK_W)], idx_v)
    pltpu.sync_copy(hid_hbm.at[pl.ds(base, _TOK_W)], rows_v)
    pltpu.async_copy(rows_v, xh_hbm.at[idx_v], sem).wait()


_ROWS_S = N_TOK // _NW


@functools.partial(
    pl.kernel, mesh=_MESH,
    out_type=jax.ShapeDtypeStruct((N_TOK, MOE_DIM), jnp.float32),
    scratch_types=[pltpu.VMEM((_ROWS_S,), jnp.int32),
                   pltpu.VMEM((_ROWS_S, MOE_DIM), jnp.float32),
                   pltpu.SemaphoreType.DMA],
)
def _sc_gather_back(dst_hbm, y_hbm, out_hbm, idx_v, rows_v, sem):
    wid = lax.axis_index("s") * _NC + lax.axis_index("c")
    base = wid * _ROWS_S
    pltpu.sync_copy(dst_hbm.at[pl.ds(base, _ROWS_S)], idx_v)
    pltpu.async_copy(y_hbm.at[idx_v], rows_v, sem).wait()
    pltpu.sync_copy(rows_v, out_hbm.at[pl.ds(base, _ROWS_S)])


IN_DIM = MOE_DIM + BEH_DIM
CAT_DIM = MOE_DIM + 128


def _wdma(wi_hbm, wo_hbm, wi_buf, wo_buf, sem_wi, sem_wo, eid, buf):
    ci = pltpu.make_async_copy(wi_hbm.at[eid], wi_buf.at[buf],
                               sem_wi.at[buf])
    co = pltpu.make_async_copy(wo_hbm.at[eid], wo_buf.at[buf],
                               sem_wo.at[buf])
    return ci, co


def _mlp_body(run_ref, isf_ref, erun_ref, nrun_ref,
              xcat_ref, beh_ref, wi_hbm, wo_hbm, y_ref,
              wi_buf, wo_buf, sem_wi, sem_wo):
    g = pl.program_id(0)
    run = run_ref[g]
    buf = lax.rem(run, 2)

    @pl.when(g == 0)
    def _():
        ci, co = _wdma(wi_hbm, wo_hbm, wi_buf, wo_buf, sem_wi, sem_wo,
                       erun_ref[0], 0)
        ci.start()
        co.start()

    @pl.when(isf_ref[g] == 1)
    def _():
        nr = run + 1

        @pl.when(nr < nrun_ref[0])
        def _():
            ci, co = _wdma(wi_hbm, wo_hbm, wi_buf, wo_buf, sem_wi, sem_wo,
                           erun_ref[nr], lax.rem(nr, 2))
            ci.start()
            co.start()

        ci, co = _wdma(wi_hbm, wo_hbm, wi_buf, wo_buf, sem_wi, sem_wo,
                       erun_ref[run], buf)
        ci.wait()
        co.wait()

    xcat = xcat_ref[...]
    xh = xcat[:, :MOE_DIM]
    wi = wi_buf[buf]
    dn = (((1,), (0,)), ((), ()))
    xb = jax.lax.dot_general(xcat[:, MOE_DIM:], beh_ref[...], dn,
                             preferred_element_type=jnp.float32)
    inter = jax.lax.dot_general(xh, wi[:MOE_DIM, :], dn,
                                preferred_element_type=jnp.float32)
    inter += jax.lax.dot_general(xb, wi[MOE_DIM:, :], dn,
                                 preferred_element_type=jnp.float32)
    inter = jnp.maximum(inter, 0.0)
    y_ref[...] = jax.lax.dot_general(inter, wo_buf[buf], dn,
                                     preferred_element_type=jnp.float32)


def _expert_mlp(run_id, is_first, e_run, num_runs, xcat, beh, Wi, Wo):
    grid_spec = pltpu.PrefetchScalarGridSpec(
        num_scalar_prefetch=4,
        grid=(NTILES,),
        in_specs=[
            pl.BlockSpec((TILE, CAT_DIM), lambda g, *_: (g, 0)),
            pl.BlockSpec((128, BEH_DIM), lambda g, *_: (0, 0)),
            pl.BlockSpec(memory_space=pl.ANY),
            pl.BlockSpec(memory_space=pl.ANY),
        ],
        out_specs=pl.BlockSpec((TILE, MOE_DIM), lambda g, *_: (g, 0)),
        scratch_shapes=[
            pltpu.VMEM((2, IN_DIM, FF_DIM), jnp.float32),
            pltpu.VMEM((2, FF_DIM, MOE_DIM), jnp.float32),
            pltpu.SemaphoreType.DMA((2,)),
            pltpu.SemaphoreType.DMA((2,)),
        ],
    )
    return pl.pallas_call(
        _mlp_body,
        grid_spec=grid_spec,
        out_shape=jax.ShapeDtypeStruct((PADDED, MOE_DIM), jnp.float32),
    )(run_id, is_first, e_run, num_runs, xcat, beh, Wi, Wo)


def kernel(hidden_states, position_index, behavior_index, Wi, Wo,
           behavior_embedding):
    pos = position_index.astype(jnp.int32)
    bidx = behavior_index.astype(jnp.int32)

    oh = (pos[:, None] == jnp.arange(NUM_EXPERTS, dtype=jnp.int32)[None, :])
    oh = oh.astype(jnp.int32)
    counts = oh.sum(axis=0)
    pad_counts = ((counts + TILE - 1) // TILE) * TILE
    ends = jnp.cumsum(pad_counts)
    pad_offset = ends - pad_counts
    rank = jnp.take_along_axis(jnp.cumsum(oh, axis=0) - oh,
                               pos[:, None], axis=1)[:, 0]
    dst_slot = pad_offset[pos] + rank
    tile_starts = jnp.arange(NTILES, dtype=jnp.int32) * TILE
    expert_of_tile = jnp.minimum(
        jnp.sum((ends[None, :] <= tile_starts[:, None]).astype(jnp.int32),
                axis=1),
        NUM_EXPERTS - 1).astype(jnp.int32)
    is_first = jnp.concatenate([jnp.ones((1,), jnp.int32),
                                (expert_of_tile[1:]
                                 != expert_of_tile[:-1]).astype(jnp.int32)])
    run_id = jnp.cumsum(is_first) - 1
    num_runs = run_id[NTILES - 1:] + 1
    e_run = jnp.zeros((NUM_EXPERTS,), jnp.int32).at[run_id].set(
        expert_of_tile)

    oh128 = (bidx[:, None] ==
             jnp.arange(128, dtype=jnp.int32)[None, :]).astype(jnp.float32)
    xcat = jnp.concatenate([hidden_states, oh128], axis=1)
    xcat_pad = _sc_scatter_pad(dst_slot, xcat)

    beh128 = jnp.zeros((128, BEH_DIM), jnp.float32
                       ).at[:NUM_BEH + 1].set(behavior_embedding)
    y = _expert_mlp(run_id, is_first, e_run, num_runs,
                    xcat_pad, beh128,
                    Wi.transpose(0, 2, 1), Wo.transpose(0, 2, 1))

    return _sc_gather_back(dst_slot, y)


# speedup vs baseline: 1.0241x; 1.0241x over previous
import functools

import jax
import jax.numpy as jnp
from jax import lax
from jax.experimental import pallas as pl
from jax.experimental.pallas import tpu as pltpu
from jax.experimental.pallas import tpu_sc as plsc

NUM_EXPERTS = 8
MOE_DIM = 768
FF_DIM = 1024
BEH_DIM = 64
NUM_BEH = 16
BEH_ROWS = 32
N_TOK = 2048
TILE = 128
NTILES = 24
PADDED = NTILES * TILE

_NC, _NS = 2, 16
_NW = _NC * _NS
_MESH = plsc.VectorSubcoreMesh(core_axis_name="c", subcore_axis_name="s",
                               num_cores=_NC, num_subcores=_NS)


_TOK_W = N_TOK // _NW


@functools.partial(
    pl.kernel, mesh=_MESH,
    out_type=jax.ShapeDtypeStruct((PADDED, MOE_DIM), jnp.float32),
    scratch_types=[pltpu.VMEM((_TOK_W,), jnp.int32),
                   pltpu.VMEM((_TOK_W, MOE_DIM), jnp.float32),
                   pltpu.SemaphoreType.DMA],
)
def _sc_scatter_pad(dst_hbm, hid_hbm, xh_hbm, idx_v, rows_v, sem):
    wid = lax.axis_index("s") * _NC + lax.axis_index("c")
    base = wid * _TOK_W
    pltpu.sync_copy(dst_hbm.at[pl.ds(base, _TOK_W)], idx_v)
    pltpu.sync_copy(hid_hbm.at[pl.ds(base, _TOK_W)], rows_v)
    pltpu.async_copy(rows_v, xh_hbm.at[idx_v], sem).wait()


_ROWS_S = N_TOK // _NW


@functools.partial(
    pl.kernel, mesh=_MESH,
    out_type=jax.ShapeDtypeStruct((N_TOK, MOE_DIM), jnp.float32),
    scratch_types=[pltpu.VMEM((_ROWS_S,), jnp.int32),
                   pltpu.VMEM((_ROWS_S, MOE_DIM), jnp.float32),
                   pltpu.SemaphoreType.DMA],
)
def _sc_gather_back(dst_hbm, y_hbm, out_hbm, idx_v, rows_v, sem):
    wid = lax.axis_index("s") * _NC + lax.axis_index("c")
    base = wid * _ROWS_S
    pltpu.sync_copy(dst_hbm.at[pl.ds(base, _ROWS_S)], idx_v)
    pltpu.async_copy(y_hbm.at[idx_v], rows_v, sem).wait()
    pltpu.sync_copy(rows_v, out_hbm.at[pl.ds(base, _ROWS_S)])


IN_DIM = MOE_DIM + BEH_DIM


def _wdma(wi_hbm, wo_hbm, wi_buf, wo_buf, sem_wi, sem_wo, eid, buf):
    ci = pltpu.make_async_copy(wi_hbm.at[eid], wi_buf.at[buf],
                               sem_wi.at[buf])
    co = pltpu.make_async_copy(wo_hbm.at[eid], wo_buf.at[buf],
                               sem_wo.at[buf])
    return ci, co


def _mlp_body(run_ref, isf_ref, erun_ref, nrun_ref,
              xh_ref, bsrc_ref, beh_ref, wi_hbm, wo_hbm, y_ref,
              wi_buf, wo_buf, sem_wi, sem_wo):
    g = pl.program_id(0)
    run = run_ref[g]
    buf = lax.rem(run, 2)

    @pl.when(g == 0)
    def _():
        ci, co = _wdma(wi_hbm, wo_hbm, wi_buf, wo_buf, sem_wi, sem_wo,
                       erun_ref[0], 0)
        ci.start()
        co.start()

    @pl.when(isf_ref[g] == 1)
    def _():
        nr = run + 1

        @pl.when(nr < nrun_ref[0])
        def _():
            ci, co = _wdma(wi_hbm, wo_hbm, wi_buf, wo_buf, sem_wi, sem_wo,
                           erun_ref[nr], lax.rem(nr, 2))
            ci.start()
            co.start()

        ci, co = _wdma(wi_hbm, wo_hbm, wi_buf, wo_buf, sem_wi, sem_wo,
                       erun_ref[run], buf)
        ci.wait()
        co.wait()

    xh = xh_ref[...]
    wi = wi_buf[buf]
    dn = (((1,), (0,)), ((), ()))
    bs = bsrc_ref[0, 0, :]
    onehot = (bs[:, None] ==
              lax.broadcasted_iota(jnp.int32, (TILE, BEH_ROWS), 1))
    xb = jax.lax.dot_general(onehot.astype(jnp.float32), beh_ref[...],
                             (((1,), (0,)), ((), ())),
                             preferred_element_type=jnp.float32)
    inter = jax.lax.dot_general(xh, wi[:MOE_DIM, :], dn,
                                preferred_element_type=jnp.float32)
    inter += jax.lax.dot_general(xb, wi[MOE_DIM:, :], dn,
                                 preferred_element_type=jnp.float32)
    inter = jnp.maximum(inter, 0.0)
    y_ref[...] = jax.lax.dot_general(inter, wo_buf[buf], dn,
                                     preferred_element_type=jnp.float32)


def _expert_mlp(run_id, is_first, e_run, num_runs, xh, bsrc, beh, Wi, Wo):
    grid_spec = pltpu.PrefetchScalarGridSpec(
        num_scalar_prefetch=4,
        grid=(NTILES,),
        in_specs=[
            pl.BlockSpec((TILE, MOE_DIM), lambda g, *_: (g, 0)),
            pl.BlockSpec((1, 1, TILE), lambda g, *_: (g, 0, 0)),
            pl.BlockSpec((BEH_ROWS, BEH_DIM), lambda g, *_: (0, 0)),
            pl.BlockSpec(memory_space=pl.ANY),
            pl.BlockSpec(memory_space=pl.ANY),
        ],
        out_specs=pl.BlockSpec((TILE, MOE_DIM), lambda g, *_: (g, 0)),
        scratch_shapes=[
            pltpu.VMEM((2, IN_DIM, FF_DIM), jnp.float32),
            pltpu.VMEM((2, FF_DIM, MOE_DIM), jnp.float32),
            pltpu.SemaphoreType.DMA((2,)),
            pltpu.SemaphoreType.DMA((2,)),
        ],
    )
    return pl.pallas_call(
        _mlp_body,
        grid_spec=grid_spec,
        out_shape=jax.ShapeDtypeStruct((PADDED, MOE_DIM), jnp.float32),
    )(run_id, is_first, e_run, num_runs, xh, bsrc, beh, Wi, Wo)


def kernel(hidden_states, position_index, behavior_index, Wi, Wo,
           behavior_embedding):
    pos = position_index.astype(jnp.int32)
    bidx = behavior_index.astype(jnp.int32)

    oh = (pos[:, None] == jnp.arange(NUM_EXPERTS, dtype=jnp.int32)[None, :])
    oh = oh.astype(jnp.int32)
    counts = oh.sum(axis=0)
    pad_counts = ((counts + TILE - 1) // TILE) * TILE
    ends = jnp.cumsum(pad_counts)
    pad_offset = ends - pad_counts
    rank = jnp.take_along_axis(jnp.cumsum(oh, axis=0) - oh,
                               pos[:, None], axis=1)[:, 0]
    dst_slot = pad_offset[pos] + rank
    bsrc_idx = jnp.zeros((PADDED,), jnp.int32).at[dst_slot].set(
        bidx).reshape(NTILES, 1, TILE)
    tile_starts = jnp.arange(NTILES, dtype=jnp.int32) * TILE
    expert_of_tile = jnp.minimum(
        jnp.sum((ends[None, :] <= tile_starts[:, None]).astype(jnp.int32),
                axis=1),
        NUM_EXPERTS - 1).astype(jnp.int32)
    is_first = jnp.concatenate([jnp.ones((1,), jnp.int32),
                                (expert_of_tile[1:]
                                 != expert_of_tile[:-1]).astype(jnp.int32)])
    run_id = jnp.cumsum(is_first) - 1
    num_runs = run_id[NTILES - 1:] + 1
    e_run = jnp.zeros((NUM_EXPERTS,), jnp.int32).at[run_id].set(
        expert_of_tile)

    xh = _sc_scatter_pad(dst_slot, hidden_states)

    beh32 = jnp.zeros((BEH_ROWS, BEH_DIM), jnp.float32
                      ).at[:NUM_BEH + 1].set(behavior_embedding)
    y = _expert_mlp(run_id, is_first, e_run, num_runs,
                    xh, bsrc_idx, beh32,
                    Wi.transpose(0, 2, 1), Wo.transpose(0, 2, 1))

    return _sc_gather_back(dst_slot, y)
